# baseline (speedup 1.0000x reference)
import functools

import jax
import jax.numpy as jnp
from jax import lax
from jax.experimental import pallas as pl
from jax.experimental.pallas import tpu as pltpu
from jax.experimental.pallas import tpu_sc as plsc

_G = 4


def _encoder_body(x_ref, c_ref, ct_ref, soft_ref, idx_ref, gidx_ref):
    G = x_ref.shape[1]
    N = x_ref.shape[2]
    CD = x_ref.shape[3]
    K = c_ref.shape[2]

    PT = 7
    NP = N // PT

    for g in range(G):
        x = x_ref[0, g]
        c = c_ref[0, g]
        ct = ct_ref[0, g]
        kio = lax.broadcasted_iota(jnp.int32, (NP, K), 1)

        idx_parts = []
        soft_parts = []
        for pt in range(PT):
            xp = x[pt * NP:(pt + 1) * NP]

            d2 = jnp.zeros((NP, K), jnp.float32)
            for dch in range(CD):
                diff = xp[:, dch:dch + 1] - ct[dch:dch + 1, :]
                d2 = d2 + diff * diff
            d = jnp.sqrt(d2)

            dmin = jnp.min(d, axis=1, keepdims=True)
            idx_parts.append(
                jnp.min(jnp.where(d == dmin, kio, K), axis=1))

            p = jnp.exp(dmin - d)
            s = jnp.sum(p, axis=1, keepdims=True)
            soft = lax.dot_general(p, c, (((1,), (0,)), ((), ())),
                                   preferred_element_type=jnp.float32)
            soft_parts.append(soft / s)

        idx = jnp.concatenate(idx_parts, axis=0)
        idx_ref[0, g, 0] = idx
        gidx_ref[0, g, 0] = idx + K * (G * pl.program_id(0) + g)
        soft_ref[0, g] = jnp.concatenate(soft_parts, axis=0)


@jax.jit
def _encode(zt, codes):
    L, N, CD = zt.shape
    _, K, _ = codes.shape
    G = _G
    codes_t = jnp.transpose(codes, (0, 2, 1))
    soft, idx, gidx = pl.pallas_call(
        _encoder_body,
        grid=(L // G,),
        in_specs=[
            pl.BlockSpec((1, G, N, CD), lambda i: (i, 0, 0, 0)),
            pl.BlockSpec((1, G, K, CD), lambda i: (i, 0, 0, 0)),
            pl.BlockSpec((1, G, CD, K), lambda i: (i, 0, 0, 0)),
        ],
        out_specs=[
            pl.BlockSpec((1, G, N, CD), lambda i: (i, 0, 0, 0)),
            pl.BlockSpec((1, G, 1, N), lambda i: (i, 0, 0, 0)),
            pl.BlockSpec((1, G, 1, N), lambda i: (i, 0, 0, 0)),
        ],
        out_shape=[
            jax.ShapeDtypeStruct((L // G, G, N, CD), jnp.float32),
            jax.ShapeDtypeStruct((L // G, G, 1, N), jnp.int32),
            jax.ShapeDtypeStruct((L // G, G, 1, N), jnp.int32),
        ],
    )(zt.reshape(L // G, G, N, CD), codes.reshape(L // G, G, K, CD),
      codes_t.reshape(L // G, G, CD, K))
    return (soft.reshape(L, N, CD), idx.reshape(L, N), gidx.reshape(L, N))


_SC_CORES = 2
_SC_SUBCORES = 16
_SC_WORKERS = _SC_CORES * _SC_SUBCORES


@functools.partial(jax.jit, static_argnames=("rows_per_worker", "row_width"))
def _sc_gather(table, qidx, rows_per_worker, row_width):
    total = qidx.shape[0]
    mesh = plsc.VectorSubcoreMesh(core_axis_name="c", subcore_axis_name="s")

    @functools.partial(
        pl.kernel,
        mesh=mesh,
        compiler_params=pltpu.CompilerParams(use_tc_tiling_on_sc=False),
        out_type=jax.ShapeDtypeStruct((total, row_width), jnp.float32),
        scratch_types=[
            pltpu.VMEM((rows_per_worker,), jnp.int32),
            pltpu.VMEM((rows_per_worker, row_width), jnp.float32),
            pltpu.SemaphoreType.DMA,
        ],
    )
    def gather_k(table_hbm, idx_hbm, out_hbm, idx_v, rows_v, sem):
        wid = lax.axis_index("s") * _SC_CORES + lax.axis_index("c")
        base = wid * rows_per_worker
        pltpu.sync_copy(idx_hbm.at[pl.ds(base, rows_per_worker)], idx_v)
        pltpu.async_copy(table_hbm.at[idx_v], rows_v, sem).wait()
        pltpu.sync_copy(rows_v, out_hbm.at[pl.ds(base, rows_per_worker)])

    return gather_k(table, qidx)


def kernel(z, codes):
    B, C, H, Wd = z.shape
    L, K, CD = codes.shape
    N = B * H * Wd
    h = jnp.transpose(z, (0, 2, 3, 1)).reshape(N, L, CD)
    zt = jnp.transpose(h, (1, 0, 2))

    soft, idx, gidx = _encode(zt, codes)

    soft_symbols = jnp.transpose(soft, (1, 0, 2)).reshape(B, H, Wd, C)
    idxes = jnp.transpose(idx, (1, 0)).reshape(B, H, Wd, L)

    q = jnp.transpose(gidx, (1, 0)).reshape(-1)
    total = N * L
    chunk = 8 * _SC_WORKERS
    padded = ((total + chunk - 1) // chunk) * chunk
    q = jnp.pad(q, (0, padded - total))
    table = codes.reshape(L * K, CD)
    rows = _sc_gather(table, q, padded // _SC_WORKERS, CD)
    hard_symbols = rows[:total].reshape(N, C).reshape(B, H, Wd, C)

    return (soft_symbols, hard_symbols, idxes)


# speedup vs baseline: 1.0734x; 1.0734x over previous
import functools

import jax
import jax.numpy as jnp
from jax import lax
from jax.experimental import pallas as pl
from jax.experimental.pallas import tpu as pltpu
from jax.experimental.pallas import tpu_sc as plsc

_G = 8


def _encoder_body(x_ref, c_ref, ct_ref, soft_ref, idx_ref, gidx_ref):
    G = x_ref.shape[1]
    N = x_ref.shape[2]
    CD = x_ref.shape[3]
    K = c_ref.shape[2]

    for g in range(G):
        x = x_ref[0, g]
        c = c_ref[0, g]
        ct = ct_ref[0, g]

        d2 = jnp.zeros((N, K), jnp.float32)
        for dch in range(CD):
            diff = x[:, dch:dch + 1] - ct[dch:dch + 1, :]
            d2 = d2 + diff * diff
        d = jnp.sqrt(d2)

        dmin = jnp.min(d, axis=1, keepdims=True)
        kio = lax.broadcasted_iota(jnp.int32, (N, K), 1)
        idx = jnp.min(jnp.where(d == dmin, kio, K), axis=1)
        idx_ref[0, g, 0] = idx
        gidx_ref[0, g, 0] = idx + K * (G * pl.program_id(0) + g)

        p = jnp.exp(dmin - d)
        s = jnp.sum(p, axis=1, keepdims=True)
        soft = lax.dot_general(p, c, (((1,), (0,)), ((), ())),
                               preferred_element_type=jnp.float32)
        soft_ref[0, g] = soft / s


@jax.jit
def _encode(zt, codes):
    L, N, CD = zt.shape
    _, K, _ = codes.shape
    G = _G
    codes_t = jnp.transpose(codes, (0, 2, 1))
    soft, idx, gidx = pl.pallas_call(
        _encoder_body,
        grid=(L // G,),
        in_specs=[
            pl.BlockSpec((1, G, N, CD), lambda i: (i, 0, 0, 0)),
            pl.BlockSpec((1, G, K, CD), lambda i: (i, 0, 0, 0)),
            pl.BlockSpec((1, G, CD, K), lambda i: (i, 0, 0, 0)),
        ],
        out_specs=[
            pl.BlockSpec((1, G, N, CD), lambda i: (i, 0, 0, 0)),
            pl.BlockSpec((1, G, 1, N), lambda i: (i, 0, 0, 0)),
            pl.BlockSpec((1, G, 1, N), lambda i: (i, 0, 0, 0)),
        ],
        out_shape=[
            jax.ShapeDtypeStruct((L // G, G, N, CD), jnp.float32),
            jax.ShapeDtypeStruct((L // G, G, 1, N), jnp.int32),
            jax.ShapeDtypeStruct((L // G, G, 1, N), jnp.int32),
        ],
    )(zt.reshape(L // G, G, N, CD), codes.reshape(L // G, G, K, CD),
      codes_t.reshape(L // G, G, CD, K))
    return (soft.reshape(L, N, CD), idx.reshape(L, N), gidx.reshape(L, N))


_SC_CORES = 2
_SC_SUBCORES = 16
_SC_WORKERS = _SC_CORES * _SC_SUBCORES


@functools.partial(jax.jit, static_argnames=("rows_per_worker", "row_width"))
def _sc_gather(table, qidx, rows_per_worker, row_width):
    total = qidx.shape[0]
    mesh = plsc.VectorSubcoreMesh(core_axis_name="c", subcore_axis_name="s")

    @functools.partial(
        pl.kernel,
        mesh=mesh,
        compiler_params=pltpu.CompilerParams(use_tc_tiling_on_sc=False),
        out_type=jax.ShapeDtypeStruct((total, row_width), jnp.float32),
        scratch_types=[
            pltpu.VMEM((rows_per_worker,), jnp.int32),
            pltpu.VMEM((rows_per_worker, row_width), jnp.float32),
            pltpu.SemaphoreType.DMA,
        ],
    )
    def gather_k(table_hbm, idx_hbm, out_hbm, idx_v, rows_v, sem):
        wid = lax.axis_index("s") * _SC_CORES + lax.axis_index("c")
        base = wid * rows_per_worker
        pltpu.sync_copy(idx_hbm.at[pl.ds(base, rows_per_worker)], idx_v)
        pltpu.async_copy(table_hbm.at[idx_v], rows_v, sem).wait()
        pltpu.sync_copy(rows_v, out_hbm.at[pl.ds(base, rows_per_worker)])

    return gather_k(table, qidx)


def kernel(z, codes):
    B, C, H, Wd = z.shape
    L, K, CD = codes.shape
    N = B * H * Wd
    h = jnp.transpose(z, (0, 2, 3, 1)).reshape(N, L, CD)
    zt = jnp.transpose(h, (1, 0, 2))

    soft, idx, gidx = _encode(zt, codes)

    soft_symbols = jnp.transpose(soft, (1, 0, 2)).reshape(B, H, Wd, C)
    idxes = jnp.transpose(idx, (1, 0)).reshape(B, H, Wd, L)

    q = jnp.transpose(gidx, (1, 0)).reshape(-1)
    total = N * L
    chunk = 8 * _SC_WORKERS
    padded = ((total + chunk - 1) // chunk) * chunk
    q = jnp.pad(q, (0, padded - total))
    table = codes.reshape(L * K, CD)
    rows = _sc_gather(table, q, padded // _SC_WORKERS, CD)
    hard_symbols = rows[:total].reshape(N, C).reshape(B, H, Wd, C)

    return (soft_symbols, hard_symbols, idxes)
